# baseline (speedup 1.0000x reference)
import functools

import jax
import jax.numpy as jnp
from jax import lax
from jax.experimental import pallas as pl
from jax.experimental.pallas import tpu as pltpu
from jax.experimental.pallas import tpu_sc as plsc

_B, _D, _HW = 16, 64, 1024
_K = 1024
_NC, _NS = 2, 16


def _vq_tc_body(z_ref, cb_ref, codes_ref, loss_ref, c2b_s, cn_s, row_s):
    b = pl.program_id(0)

    @pl.when(b == 0)
    def _():
        c = cb_ref[...]
        c2b_s[...] = (2.0 * c).astype(jnp.bfloat16)
        cn_s[...] = jnp.sum(c * c, axis=1, keepdims=True)
        row_s[...] = lax.broadcasted_iota(
            jnp.int32, (_K, _HW), 0).astype(jnp.float32)

    z = z_ref[0]
    s2 = jnp.dot(c2b_s[...], z.astype(jnp.bfloat16),
                 preferred_element_type=jnp.float32)
    dist = cn_s[...] - s2
    m = jnp.min(dist, axis=0)
    codes_f = jnp.min(jnp.where(dist == m[None, :], row_s[...],
                                jnp.float32(2.0e9)), axis=0)
    codes_ref[0, 0, :] = codes_f.astype(jnp.int32)
    part = jnp.reshape(jnp.sum(m) + jnp.sum(z * z), (1, 1))

    @pl.when(b == 0)
    def _():
        loss_ref[...] = part

    @pl.when(b > 0)
    def _():
        loss_ref[...] += part


def _vq_sc_gather(cb_hbm, codes_hbm, out4_hbm, cb_v, codes_v, out_v, sem):
    b = lax.axis_index("s")
    half = lax.axis_index("c")
    base_d = half * (_D // 2)
    pltpu.sync_copy(cb_hbm.at[:, pl.ds(base_d, _D // 2)], cb_v)
    pltpu.sync_copy(codes_hbm.at[pl.ds(b * _HW, _HW)], codes_v)

    nd = _D // 2
    chunk = nd // 4
    copies = []
    for cstart in range(0, nd, chunk):
        @plsc.parallel_loop(0, _HW // 16, unroll=2)
        def _(j):
            code16 = codes_v[pl.ds(j * 16, 16)]
            hh = j // 2
            w0 = (j % 2) * 16
            for dd in range(cstart, cstart + chunk):
                out_v[dd, hh, pl.ds(w0, 16)] = plsc.load_gather(
                    cb_v, [code16, jnp.full((16,), dd, jnp.int32)])
        copies.append(pltpu.async_copy(
            out_v.at[pl.ds(cstart, chunk)],
            out4_hbm.at[b, pl.ds(base_d + cstart, chunk)], sem))
    for cp in copies:
        cp.wait()


def _sc_gather_call(codebook, codes_flat):
    mesh = plsc.VectorSubcoreMesh(core_axis_name="c", subcore_axis_name="s")
    fn = functools.partial(
        pl.kernel,
        mesh=mesh,
        out_type=jax.ShapeDtypeStruct((_B, _D, 32, 32), jnp.float32),
        scratch_types=[
            pltpu.VMEM((_K, _D // 2), jnp.float32),
            pltpu.VMEM((_HW,), jnp.int32),
            pltpu.VMEM((_D // 2, 32, 32), jnp.float32),
            pltpu.SemaphoreType.DMA,
        ],
        compiler_params=pltpu.CompilerParams(
            needs_layout_passes=False, use_tc_tiling_on_sc=False),
    )(_vq_sc_gather)
    return fn(codebook, codes_flat)


def kernel(z_e, codebook):
    B, D, H, W = z_e.shape
    z3 = z_e.reshape(B, D, H * W)
    codes3, loss_arr = pl.pallas_call(
        _vq_tc_body,
        grid=(B,),
        in_specs=[
            pl.BlockSpec((1, D, H * W), lambda b: (b, 0, 0)),
            pl.BlockSpec((_K, D), lambda b: (0, 0)),
        ],
        out_specs=[
            pl.BlockSpec((1, 1, H * W), lambda b: (b, 0, 0)),
            pl.BlockSpec((1, 1), lambda b: (0, 0)),
        ],
        out_shape=[
            jax.ShapeDtypeStruct((B, 1, H * W), jnp.int32),
            jax.ShapeDtypeStruct((1, 1), jnp.float32),
        ],
        scratch_shapes=[
            pltpu.VMEM((_K, D), jnp.bfloat16),
            pltpu.VMEM((_K, 1), jnp.float32),
            pltpu.VMEM((_K, H * W), jnp.float32),
        ],
        compiler_params=pltpu.CompilerParams(
            dimension_semantics=("arbitrary",)),
    )(z3, codebook)

    codes_flat = codes3.reshape(B * H * W)
    z_q = _sc_gather_call(codebook, codes_flat)
    indices = codes3.reshape(B, H, W)
    vq_loss = (1.25 / (B * D * H * W)) * loss_arr[0, 0]
    return (z_q, indices, vq_loss)


# speedup vs baseline: 1.2227x; 1.2227x over previous
import functools

import jax
import jax.numpy as jnp
from jax import lax
from jax.experimental import pallas as pl
from jax.experimental.pallas import tpu as pltpu
from jax.experimental.pallas import tpu_sc as plsc

_B, _D, _HW = 16, 64, 1024
_K = 1024
_NC, _NS = 2, 16


def _vq_tc_body(z_ref, cb_ref, codes_ref, loss_ref, c2b_s, cn_s, row_s):
    b = pl.program_id(0)

    @pl.when(b == 0)
    def _():
        c = cb_ref[...]
        c2b_s[...] = (2.0 * c).astype(jnp.bfloat16)
        cn_s[...] = jnp.sum(c * c, axis=1, keepdims=True)
        row_s[...] = lax.broadcasted_iota(
            jnp.int32, (_K, _HW), 0).astype(jnp.float32)

    z = z_ref[0]
    s2 = jnp.dot(c2b_s[...], z.astype(jnp.bfloat16),
                 preferred_element_type=jnp.float32)
    dist = cn_s[...] - s2
    m = jnp.min(dist, axis=0)
    codes_f = jnp.min(jnp.where(dist == m[None, :], row_s[...],
                                jnp.float32(2.0e9)), axis=0)
    codes_ref[0, 0, :] = codes_f.astype(jnp.int32)
    part = jnp.reshape(jnp.sum(m) + jnp.sum(z * z), (1, 1))

    @pl.when(b == 0)
    def _():
        loss_ref[...] = part

    @pl.when(b > 0)
    def _():
        loss_ref[...] += part


def _vq_sc_gather(cb_hbm, codes_hbm, out_hbm, cb_v, codes_v, out_v, sem):
    b = lax.axis_index("s")
    half = lax.axis_index("c")
    base_d = half * (_D // 2)
    in0 = pltpu.async_copy(cb_hbm.at[:, pl.ds(base_d, _D // 2)], cb_v, sem)
    in1 = pltpu.async_copy(codes_hbm.at[pl.ds(b * _HW, _HW)], codes_v, sem)
    in0.wait()
    in1.wait()

    nd = _D // 2
    chunk = nd // 8
    copies = []
    for cstart in range(0, nd, chunk):
        @plsc.parallel_loop(0, _HW // 16, unroll=4)
        def _(j):
            code16 = codes_v[pl.ds(j * 16, 16)]
            for dd in range(cstart, cstart + chunk):
                out_v[dd, pl.ds(j * 16, 16)] = plsc.load_gather(
                    cb_v, [code16, jnp.full((16,), dd, jnp.int32)])
        copies.append(pltpu.async_copy(
            out_v.at[pl.ds(cstart, chunk)],
            out_hbm.at[b, pl.ds(base_d + cstart, chunk)], sem))
    for cp in copies:
        cp.wait()


def _sc_gather_call(codebook, codes_flat):
    mesh = plsc.VectorSubcoreMesh(core_axis_name="c", subcore_axis_name="s")
    fn = functools.partial(
        pl.kernel,
        mesh=mesh,
        out_type=jax.ShapeDtypeStruct((_B, _D, _HW), jnp.float32),
        scratch_types=[
            pltpu.VMEM((_K, _D // 2), jnp.float32),
            pltpu.VMEM((_HW,), jnp.int32),
            pltpu.VMEM((_D // 2, _HW), jnp.float32),
            pltpu.SemaphoreType.DMA,
        ],
        compiler_params=pltpu.CompilerParams(
            needs_layout_passes=False, use_tc_tiling_on_sc=False),
    )(_vq_sc_gather)
    return fn(codebook, codes_flat)


def kernel(z_e, codebook):
    B, D, H, W = z_e.shape
    z3 = z_e.reshape(B, D, H * W)
    codes3, loss_arr = pl.pallas_call(
        _vq_tc_body,
        grid=(B,),
        in_specs=[
            pl.BlockSpec((1, D, H * W), lambda b: (b, 0, 0)),
            pl.BlockSpec((_K, D), lambda b: (0, 0)),
        ],
        out_specs=[
            pl.BlockSpec((1, 1, H * W), lambda b: (b, 0, 0)),
            pl.BlockSpec((1, 1), lambda b: (0, 0)),
        ],
        out_shape=[
            jax.ShapeDtypeStruct((B, 1, H * W), jnp.int32),
            jax.ShapeDtypeStruct((1, 1), jnp.float32),
        ],
        scratch_shapes=[
            pltpu.VMEM((_K, D), jnp.bfloat16),
            pltpu.VMEM((_K, 1), jnp.float32),
            pltpu.VMEM((_K, H * W), jnp.float32),
        ],
        compiler_params=pltpu.CompilerParams(
            dimension_semantics=("arbitrary",)),
    )(z3, codebook)

    codes_flat = codes3.reshape(B * H * W)
    zq3 = _sc_gather_call(codebook, codes_flat)
    z_q = zq3.reshape(B, D, H, W)
    indices = codes3.reshape(B, H, W)
    vq_loss = (1.25 / (B * D * H * W)) * loss_arr[0, 0]
    return (z_q, indices, vq_loss)


# speedup vs baseline: 1.3568x; 1.1097x over previous
import functools

import jax
import jax.numpy as jnp
from jax import lax
from jax.experimental import pallas as pl
from jax.experimental.pallas import tpu as pltpu
from jax.experimental.pallas import tpu_sc as plsc

_B, _D, _HW = 16, 64, 1024
_K = 1024
_NC, _NS = 2, 16


def _vq_tc_body(z_ref, cb_ref, codes_ref, loss_ref, c2b_s, cn_s, row_s):
    b = pl.program_id(0)

    @pl.when(b == 0)
    def _():
        c = cb_ref[...]
        c2b_s[...] = (2.0 * c).astype(jnp.bfloat16)
        cn_s[...] = jnp.sum(c * c, axis=1, keepdims=True)
        row_s[...] = lax.broadcasted_iota(
            jnp.int32, (_K, _HW), 0).astype(jnp.float32)

    z = z_ref[0]
    s2 = jnp.dot(c2b_s[...], z.astype(jnp.bfloat16),
                 preferred_element_type=jnp.float32)
    dist = cn_s[...] - s2
    m = jnp.min(dist, axis=0)
    codes_f = jnp.min(jnp.where(dist == m[None, :], row_s[...],
                                jnp.float32(2.0e9)), axis=0)
    codes_ref[0, 0, :] = codes_f.astype(jnp.int32)
    part = jnp.reshape(jnp.sum(m) + jnp.sum(z * z), (1, 1))

    @pl.when(b == 0)
    def _():
        loss_ref[...] = part

    @pl.when(b > 0)
    def _():
        loss_ref[...] += part


def _vq_sc_gather(cb_hbm, codes_hbm, out_hbm, cb_v, codes_v, out_v, sem):
    wid = lax.axis_index("s") * 2 + lax.axis_index("c")
    b = wid // 4
    nd = _D // 4
    base_d = (wid % 4) * nd
    in0 = pltpu.async_copy(cb_hbm.at[:, pl.ds(base_d, nd)], cb_v, sem)
    in1 = pltpu.async_copy(codes_hbm.at[pl.ds(b * _HW, _HW)], codes_v, sem)
    in0.wait()
    in1.wait()

    chunk = nd // 4
    copies = []
    for cstart in range(0, nd, chunk):
        @plsc.parallel_loop(0, _HW // 16, unroll=4)
        def _(j):
            code16 = codes_v[pl.ds(j * 16, 16)]
            for dd in range(cstart, cstart + chunk):
                out_v[dd, pl.ds(j * 16, 16)] = plsc.load_gather(
                    cb_v, [code16, jnp.full((16,), dd, jnp.int32)])
        copies.append(pltpu.async_copy(
            out_v.at[pl.ds(cstart, chunk)],
            out_hbm.at[b, pl.ds(base_d + cstart, chunk)], sem))
    for cp in copies:
        cp.wait()


def _sc_gather_call(codebook, codes_flat):
    mesh = plsc.VectorSubcoreMesh(core_axis_name="c", subcore_axis_name="s")
    fn = functools.partial(
        pl.kernel,
        mesh=mesh,
        out_type=jax.ShapeDtypeStruct((_B // 2, _D, _HW), jnp.float32),
        scratch_types=[
            pltpu.VMEM((_K, _D // 4), jnp.float32),
            pltpu.VMEM((_HW,), jnp.int32),
            pltpu.VMEM((_D // 4, _HW), jnp.float32),
            pltpu.SemaphoreType.DMA,
        ],
        compiler_params=pltpu.CompilerParams(
            needs_layout_passes=False, use_tc_tiling_on_sc=False),
    )(_vq_sc_gather)
    return fn(codebook, codes_flat)


def _tc_call(z3, codebook, off, nb):
    return pl.pallas_call(
        _vq_tc_body,
        grid=(nb,),
        in_specs=[
            pl.BlockSpec((1, _D, _HW), lambda b: (b + off, 0, 0)),
            pl.BlockSpec((_K, _D), lambda b: (0, 0)),
        ],
        out_specs=[
            pl.BlockSpec((1, 1, _HW), lambda b: (b, 0, 0)),
            pl.BlockSpec((1, 1), lambda b: (0, 0)),
        ],
        out_shape=[
            jax.ShapeDtypeStruct((nb, 1, _HW), jnp.int32),
            jax.ShapeDtypeStruct((1, 1), jnp.float32),
        ],
        scratch_shapes=[
            pltpu.VMEM((_K, _D), jnp.bfloat16),
            pltpu.VMEM((_K, 1), jnp.float32),
            pltpu.VMEM((_K, _HW), jnp.float32),
        ],
        compiler_params=pltpu.CompilerParams(
            dimension_semantics=("arbitrary",)),
    )(z3, codebook)


def kernel(z_e, codebook):
    B, D, H, W = z_e.shape
    z3 = z_e.reshape(B, D, H * W)
    nb = B // 2
    codes_a, loss_a = _tc_call(z3, codebook, 0, nb)
    zq_a = _sc_gather_call(codebook, codes_a.reshape(nb * H * W))
    codes_b, loss_b = _tc_call(z3, codebook, nb, nb)
    zq_b = _sc_gather_call(codebook, codes_b.reshape(nb * H * W))
    z_q = jnp.concatenate([zq_a, zq_b], axis=0).reshape(B, D, H, W)
    indices = jnp.concatenate([codes_a, codes_b], axis=0).reshape(B, H, W)
    vq_loss = (1.25 / (B * D * H * W)) * (loss_a[0, 0] + loss_b[0, 0])
    return (z_q, indices, vq_loss)


# speedup vs baseline: 1.4221x; 1.0481x over previous
import functools

import jax
import jax.numpy as jnp
from jax import lax
from jax.experimental import pallas as pl
from jax.experimental.pallas import tpu as pltpu
from jax.experimental.pallas import tpu_sc as plsc

_B, _D, _HW = 16, 64, 1024
_K = 1024
_NC, _NS = 2, 16


def _vq_tc_body(z_ref, cb_ref, codes_ref, loss_ref, c2b_s, cn_s):
    b = pl.program_id(0)

    @pl.when(b == 0)
    def _():
        c = cb_ref[...]
        c2b_s[...] = (2.0 * c).astype(jnp.bfloat16)
        cn_s[...] = jnp.sum(c * c, axis=1, keepdims=True)

    z = z_ref[0]
    s2 = jnp.dot(c2b_s[...], z.astype(jnp.bfloat16),
                 preferred_element_type=jnp.float32)
    dist = cn_s[...] - s2
    m = jnp.min(dist, axis=0)
    codes_ref[0, 0, :] = jnp.argmin(dist, axis=0).astype(jnp.int32)
    part = jnp.reshape(jnp.sum(m) + jnp.sum(z * z), (1, 1))

    @pl.when(b == 0)
    def _():
        loss_ref[...] = part

    @pl.when(b > 0)
    def _():
        loss_ref[...] += part


def _vq_sc_gather(cb_hbm, codes_hbm, out_hbm, cb_v, codes_v, out_v, sem):
    wid = lax.axis_index("s") * 2 + lax.axis_index("c")
    b = wid // 4
    nd = _D // 4
    base_d = (wid % 4) * nd
    in0 = pltpu.async_copy(cb_hbm.at[:, pl.ds(base_d, nd)], cb_v, sem)
    in1 = pltpu.async_copy(codes_hbm.at[pl.ds(b * _HW, _HW)], codes_v, sem)
    in0.wait()
    in1.wait()

    chunk = nd // 4
    copies = []
    for cstart in range(0, nd, chunk):
        @plsc.parallel_loop(0, _HW // 16, unroll=4)
        def _(j):
            code16 = codes_v[pl.ds(j * 16, 16)]
            for dd in range(cstart, cstart + chunk):
                out_v[dd, pl.ds(j * 16, 16)] = plsc.load_gather(
                    cb_v, [code16, jnp.full((16,), dd, jnp.int32)])
        copies.append(pltpu.async_copy(
            out_v.at[pl.ds(cstart, chunk)],
            out_hbm.at[b, pl.ds(base_d + cstart, chunk)], sem))
    for cp in copies:
        cp.wait()


def _sc_gather_call(codebook, codes_flat):
    mesh = plsc.VectorSubcoreMesh(core_axis_name="c", subcore_axis_name="s")
    fn = functools.partial(
        pl.kernel,
        mesh=mesh,
        out_type=jax.ShapeDtypeStruct((_B // 2, _D, _HW), jnp.float32),
        scratch_types=[
            pltpu.VMEM((_K, _D // 4), jnp.float32),
            pltpu.VMEM((_HW,), jnp.int32),
            pltpu.VMEM((_D // 4, _HW), jnp.float32),
            pltpu.SemaphoreType.DMA,
        ],
        compiler_params=pltpu.CompilerParams(
            needs_layout_passes=False, use_tc_tiling_on_sc=False),
    )(_vq_sc_gather)
    return fn(codebook, codes_flat)


def _tc_call(z3, codebook, off, nb):
    return pl.pallas_call(
        _vq_tc_body,
        grid=(nb,),
        in_specs=[
            pl.BlockSpec((1, _D, _HW), lambda b: (b + off, 0, 0)),
            pl.BlockSpec((_K, _D), lambda b: (0, 0)),
        ],
        out_specs=[
            pl.BlockSpec((1, 1, _HW), lambda b: (b, 0, 0)),
            pl.BlockSpec((1, 1), lambda b: (0, 0)),
        ],
        out_shape=[
            jax.ShapeDtypeStruct((nb, 1, _HW), jnp.int32),
            jax.ShapeDtypeStruct((1, 1), jnp.float32),
        ],
        scratch_shapes=[
            pltpu.VMEM((_K, _D), jnp.bfloat16),
            pltpu.VMEM((_K, 1), jnp.float32),
        ],
        compiler_params=pltpu.CompilerParams(
            dimension_semantics=("arbitrary",)),
    )(z3, codebook)


def kernel(z_e, codebook):
    B, D, H, W = z_e.shape
    z3 = z_e.reshape(B, D, H * W)
    nb = B // 2
    codes_a, loss_a = _tc_call(z3, codebook, 0, nb)
    zq_a = _sc_gather_call(codebook, codes_a.reshape(nb * H * W))
    codes_b, loss_b = _tc_call(z3, codebook, nb, nb)
    zq_b = _sc_gather_call(codebook, codes_b.reshape(nb * H * W))
    z_q = jnp.concatenate([zq_a, zq_b], axis=0).reshape(B, D, H, W)
    indices = jnp.concatenate([codes_a, codes_b], axis=0).reshape(B, H, W)
    vq_loss = (1.25 / (B * D * H * W)) * (loss_a[0, 0] + loss_b[0, 0])
    return (z_q, indices, vq_loss)


# speedup vs baseline: 1.4516x; 1.0207x over previous
import functools

import jax
import jax.numpy as jnp
from jax import lax
from jax.experimental import pallas as pl
from jax.experimental.pallas import tpu as pltpu
from jax.experimental.pallas import tpu_sc as plsc

_B, _D, _HW = 16, 64, 1024
_K = 1024
_NC, _NS = 2, 16


def _vq_tc_body(z_ref, cb_ref, codes_ref, loss_ref, c2b_s, cn_s):
    b = pl.program_id(0)

    @pl.when(b == 0)
    def _():
        c = cb_ref[...]
        c2b_s[...] = (2.0 * c).astype(jnp.bfloat16)
        cn_s[...] = jnp.sum(c * c, axis=1, keepdims=True)

    z = z_ref[0]
    s2 = jnp.dot(c2b_s[...], z.astype(jnp.bfloat16),
                 preferred_element_type=jnp.float32)
    dist = cn_s[...] - s2
    m = jnp.min(dist, axis=0)
    codes_ref[0, 0, :] = jnp.argmin(dist, axis=0).astype(jnp.int32)
    part = jnp.reshape(jnp.sum(m) + jnp.sum(z * z), (1, 1))

    @pl.when(b == 0)
    def _():
        loss_ref[...] = part

    @pl.when(b > 0)
    def _():
        loss_ref[...] += part


def _vq_sc_gather(off, cb_hbm, codes_hbm, out_hbm, cb_v, codes_v, out_v,
                  sem):
    wid = lax.axis_index("s") * 2 + lax.axis_index("c")
    b = wid // 4 + off
    nd = _D // 4
    base_d = (wid % 4) * nd
    in0 = pltpu.async_copy(cb_hbm.at[:, pl.ds(base_d, nd)], cb_v, sem)
    in1 = pltpu.async_copy(
        codes_hbm.at[pl.ds((b - off) * _HW, _HW)], codes_v, sem)
    in0.wait()
    in1.wait()

    chunk = nd // 4
    copies = []
    for cstart in range(0, nd, chunk):
        @plsc.parallel_loop(0, _HW // 16, unroll=4)
        def _(j):
            code16 = codes_v[pl.ds(j * 16, 16)]
            for dd in range(cstart, cstart + chunk):
                out_v[dd, pl.ds(j * 16, 16)] = plsc.load_gather(
                    cb_v, [code16, jnp.full((16,), dd, jnp.int32)])
        copies.append(pltpu.async_copy(
            out_v.at[pl.ds(cstart, chunk)],
            out_hbm.at[b, pl.ds(base_d + cstart, chunk)], sem))
    for cp in copies:
        cp.wait()


def _sc_gather_call(codebook, codes_flat, out_ref, off):
    mesh = plsc.VectorSubcoreMesh(core_axis_name="c", subcore_axis_name="s")
    fn = functools.partial(
        pl.kernel,
        mesh=mesh,
        out_type=(),
        scratch_types=[
            pltpu.VMEM((_K, _D // 4), jnp.float32),
            pltpu.VMEM((_HW,), jnp.int32),
            pltpu.VMEM((_D // 4, _HW), jnp.float32),
            pltpu.SemaphoreType.DMA,
        ],
        compiler_params=pltpu.CompilerParams(
            needs_layout_passes=False, use_tc_tiling_on_sc=False),
    )(functools.partial(_vq_sc_gather, off))
    fn(codebook, codes_flat, out_ref)


def _tc_call(z3, codebook, off, nb):
    return pl.pallas_call(
        _vq_tc_body,
        grid=(nb,),
        in_specs=[
            pl.BlockSpec((1, _D, _HW), lambda b: (b + off, 0, 0)),
            pl.BlockSpec((_K, _D), lambda b: (0, 0)),
        ],
        out_specs=[
            pl.BlockSpec((1, 1, _HW), lambda b: (b, 0, 0)),
            pl.BlockSpec((1, 1), lambda b: (0, 0)),
        ],
        out_shape=[
            jax.ShapeDtypeStruct((nb, 1, _HW), jnp.int32),
            jax.ShapeDtypeStruct((1, 1), jnp.float32),
        ],
        scratch_shapes=[
            pltpu.VMEM((_K, _D), jnp.bfloat16),
            pltpu.VMEM((_K, 1), jnp.float32),
        ],
        compiler_params=pltpu.CompilerParams(
            dimension_semantics=("arbitrary",)),
    )(z3, codebook)


def kernel(z_e, codebook):
    B, D, H, W = z_e.shape
    z3 = z_e.reshape(B, D, H * W)
    nb = B // 2
    zq_ref = jax.new_ref(jnp.zeros((B, D, H * W), jnp.float32))
    codes_a, loss_a = _tc_call(z3, codebook, 0, nb)
    _sc_gather_call(codebook, codes_a.reshape(nb * H * W), zq_ref, 0)
    codes_b, loss_b = _tc_call(z3, codebook, nb, nb)
    _sc_gather_call(codebook, codes_b.reshape(nb * H * W), zq_ref, nb)
    z_q = zq_ref[...].reshape(B, D, H, W)
    indices = jnp.concatenate([codes_a, codes_b], axis=0).reshape(B, H, W)
    vq_loss = (1.25 / (B * D * H * W)) * (loss_a[0, 0] + loss_b[0, 0])
    return (z_q, indices, vq_loss)


# speedup vs baseline: 1.5042x; 1.0363x over previous
import functools

import jax
import jax.numpy as jnp
from jax import lax
from jax.experimental import pallas as pl
from jax.experimental.pallas import tpu as pltpu
from jax.experimental.pallas import tpu_sc as plsc

_B, _D, _HW = 16, 64, 1024
_K = 1024
_NC, _NS = 2, 16


def _vq_tc_body(z_ref, cb_ref, codes_ref, loss_ref, c2b_s, cn_s):
    b = pl.program_id(0)

    @pl.when(b == 0)
    def _():
        c = cb_ref[...]
        c2b_s[...] = (2.0 * c).astype(jnp.bfloat16)
        cn_s[...] = jnp.sum(c * c, axis=1, keepdims=True)

    z = z_ref[0]
    s2 = jnp.dot(c2b_s[...], z.astype(jnp.bfloat16),
                 preferred_element_type=jnp.float32)
    dist = cn_s[...] - s2
    m = jnp.min(dist, axis=0)
    codes_ref[0, 0, :] = jnp.argmin(dist, axis=0).astype(jnp.int32)
    part = jnp.reshape(jnp.sum(m) + jnp.sum(z * z), (1, 1))

    @pl.when(b == 0)
    def _():
        loss_ref[...] = part

    @pl.when(b > 0)
    def _():
        loss_ref[...] += part


def _vq_sc_gather(off, cb_hbm, codes_hbm, out_hbm, cb_v, codes_v, out_v,
                  sem):
    wid = lax.axis_index("s") * 2 + lax.axis_index("c")
    b = wid // 4 + off
    nd = _D // 4
    base_d = (wid % 4) * nd
    in0 = pltpu.async_copy(cb_hbm.at[:, pl.ds(base_d, nd)], cb_v, sem)
    in1 = pltpu.async_copy(
        codes_hbm.at[pl.ds((b - off) * _HW, _HW)], codes_v, sem)
    in0.wait()
    in1.wait()

    chunk = nd // 4
    copies = []
    for cstart in range(0, nd, chunk):
        @plsc.parallel_loop(0, _HW // 16, unroll=4)
        def _(j):
            code16 = codes_v[pl.ds(j * 16, 16)]
            for dd in range(cstart, cstart + chunk):
                out_v[dd, pl.ds(j * 16, 16)] = plsc.load_gather(
                    cb_v, [code16, jnp.full((16,), dd, jnp.int32)])
        copies.append(pltpu.async_copy(
            out_v.at[pl.ds(cstart, chunk)],
            out_hbm.at[b, pl.ds(base_d + cstart, chunk)], sem))
    for cp in copies:
        cp.wait()


def _sc_gather_call(codebook, codes_flat, out_ref, off):
    mesh = plsc.VectorSubcoreMesh(core_axis_name="c", subcore_axis_name="s")
    fn = functools.partial(
        pl.kernel,
        mesh=mesh,
        out_type=(),
        scratch_types=[
            pltpu.VMEM((_K, _D // 4), jnp.float32),
            pltpu.VMEM((_HW,), jnp.int32),
            pltpu.VMEM((_D // 4, _HW), jnp.float32),
            pltpu.SemaphoreType.DMA,
        ],
        compiler_params=pltpu.CompilerParams(
            needs_layout_passes=False, use_tc_tiling_on_sc=False),
    )(functools.partial(_vq_sc_gather, off))
    fn(codebook, codes_flat, out_ref)


def _tc_call(z3, codebook, off, nb):
    return pl.pallas_call(
        _vq_tc_body,
        grid=(nb,),
        in_specs=[
            pl.BlockSpec((1, _D, _HW), lambda b: (b + off, 0, 0)),
            pl.BlockSpec((_K, _D), lambda b: (0, 0)),
        ],
        out_specs=[
            pl.BlockSpec((1, 1, _HW), lambda b: (b, 0, 0)),
            pl.BlockSpec((1, 1), lambda b: (0, 0)),
        ],
        out_shape=[
            jax.ShapeDtypeStruct((nb, 1, _HW), jnp.int32),
            jax.ShapeDtypeStruct((1, 1), jnp.float32),
        ],
        scratch_shapes=[
            pltpu.VMEM((_K, _D), jnp.bfloat16),
            pltpu.VMEM((_K, 1), jnp.float32),
        ],
        compiler_params=pltpu.CompilerParams(
            dimension_semantics=("arbitrary",)),
    )(z3, codebook)


def kernel(z_e, codebook):
    B, D, H, W = z_e.shape
    z3 = z_e.reshape(B, D, H * W)
    nb = B // 2
    zq_ref = jax.new_ref(lax.empty((B, D, H * W), jnp.float32))
    codes_a, loss_a = _tc_call(z3, codebook, 0, nb)
    _sc_gather_call(codebook, codes_a.reshape(nb * H * W), zq_ref, 0)
    codes_b, loss_b = _tc_call(z3, codebook, nb, nb)
    _sc_gather_call(codebook, codes_b.reshape(nb * H * W), zq_ref, nb)
    z_q = zq_ref[...].reshape(B, D, H, W)
    indices = jnp.concatenate([codes_a, codes_b], axis=0).reshape(B, H, W)
    vq_loss = (1.25 / (B * D * H * W)) * (loss_a[0, 0] + loss_b[0, 0])
    return (z_q, indices, vq_loss)


# speedup vs baseline: 1.5065x; 1.0015x over previous
import functools

import jax
import jax.numpy as jnp
from jax import lax
from jax.experimental import pallas as pl
from jax.experimental.pallas import tpu as pltpu
from jax.experimental.pallas import tpu_sc as plsc

_B, _D, _HW = 16, 64, 1024
_K = 1024


def _vq_tc_body(z_ref, cb_ref, codes_ref, loss_ref, c2b_s, cn_s):
    b = pl.program_id(0)

    @pl.when(b == 0)
    def _():
        c = cb_ref[...]
        c2b_s[...] = (2.0 * c).astype(jnp.bfloat16)
        cn_s[...] = jnp.sum(c * c, axis=1, keepdims=True)

    z = z_ref[0]
    s2 = jnp.dot(c2b_s[...], z.astype(jnp.bfloat16),
                 preferred_element_type=jnp.float32)
    dist = cn_s[...] - s2
    m = jnp.min(dist, axis=0)
    codes_ref[0, 0, :] = jnp.argmin(dist, axis=0).astype(jnp.int32)
    part = jnp.reshape(jnp.sum(m) + jnp.sum(z * z), (1, 1))

    @pl.when(b == 0)
    def _():
        loss_ref[...] = part

    @pl.when(b > 0)
    def _():
        loss_ref[...] += part


def _vq_sc_gather(off, cb_hbm, codes_hbm, out_hbm, cb_v, codes_v, out_v,
                  sem):
    wid = lax.axis_index("s") * 2 + lax.axis_index("c")
    b = wid // 4 + off
    nd = _D // 4
    base_d = (wid % 4) * nd
    in0 = pltpu.async_copy(cb_hbm.at[:, pl.ds(base_d, nd)], cb_v, sem)
    in1 = pltpu.async_copy(
        codes_hbm.at[pl.ds((b - off) * _HW, _HW)], codes_v, sem)
    in0.wait()
    in1.wait()

    chunk = nd // 4
    copies = []
    for cstart in range(0, nd, chunk):
        @plsc.parallel_loop(0, _HW // 16, unroll=4)
        def _(j):
            code16 = codes_v[pl.ds(j * 16, 16)]
            for dd in range(cstart, cstart + chunk):
                out_v[dd, pl.ds(j * 16, 16)] = plsc.load_gather(
                    cb_v, [code16, jnp.full((16,), dd, jnp.int32)])
        copies.append(pltpu.async_copy(
            out_v.at[pl.ds(cstart, chunk)],
            out_hbm.at[b, pl.ds(base_d + cstart, chunk)], sem))
    for cp in copies:
        cp.wait()


def _sc_gather_call(codebook, codes_flat, out_ref, off):
    mesh = plsc.VectorSubcoreMesh(core_axis_name="c", subcore_axis_name="s")
    fn = functools.partial(
        pl.kernel,
        mesh=mesh,
        out_type=(),
        scratch_types=[
            pltpu.VMEM((_K, _D // 4), jnp.float32),
            pltpu.VMEM((_HW,), jnp.int32),
            pltpu.VMEM((_D // 4, _HW), jnp.float32),
            pltpu.SemaphoreType.DMA,
        ],
        compiler_params=pltpu.CompilerParams(
            needs_layout_passes=False, use_tc_tiling_on_sc=False),
    )(functools.partial(_vq_sc_gather, off))
    fn(codebook, codes_flat, out_ref)


def _tc_call(z3, codebook, off, nb):
    return pl.pallas_call(
        _vq_tc_body,
        grid=(nb,),
        in_specs=[
            pl.BlockSpec((1, _D, _HW), lambda b: (b + off, 0, 0)),
            pl.BlockSpec((_K, _D), lambda b: (0, 0)),
        ],
        out_specs=[
            pl.BlockSpec((1, 1, _HW), lambda b: (b, 0, 0)),
            pl.BlockSpec((1, 1), lambda b: (0, 0)),
        ],
        out_shape=[
            jax.ShapeDtypeStruct((nb, 1, _HW), jnp.int32),
            jax.ShapeDtypeStruct((1, 1), jnp.float32),
        ],
        scratch_shapes=[
            pltpu.VMEM((_K, _D), jnp.bfloat16),
            pltpu.VMEM((_K, 1), jnp.float32),
        ],
        compiler_params=pltpu.CompilerParams(
            dimension_semantics=("arbitrary",)),
    )(z3, codebook)


def kernel(z_e, codebook):
    B, D, H, W = z_e.shape
    z3 = z_e.reshape(B, D, H * W)
    nb = B // 2
    zq_ref = jax.new_ref(lax.empty((B, D, H * W), jnp.float32))
    codes_a, loss_a = _tc_call(z3, codebook, 0, nb)
    _sc_gather_call(codebook, codes_a.reshape(nb * H * W), zq_ref, 0)
    codes_b, loss_b = _tc_call(z3, codebook, nb, nb)
    _sc_gather_call(codebook, codes_b.reshape(nb * H * W), zq_ref, nb)
    z_q = zq_ref[...].reshape(B, D, H, W)
    indices = jnp.concatenate([codes_a, codes_b], axis=0).reshape(B, H, W)
    vq_loss = (1.25 / (B * D * H * W)) * (loss_a[0, 0] + loss_b[0, 0])
    return (z_q, indices, vq_loss)
